# Initial kernel scaffold; baseline (speedup 1.0000x reference)
import functools

import jax
import jax.numpy as jnp
import numpy as np
from jax.experimental import pallas as pl

_NB, _NC, _A = 4, 192, 56
_NPIX = _A * _A
_BINS = 11
_NLAYERS = 10
_NGROUPS = 5
_R = 240


def _border_masks():
    m1 = np.ones((_A, _A), np.float32)
    m1[0, :] = 0.0
    m1[-1, :] = 0.0
    m1[:, 0] = 0.0
    m1[:, -1] = 0.0
    m2 = np.ones((_A, _A), np.float32)
    m2[0:2, :] = 0.0
    m2[-2:, :] = 0.0
    m2[:, 0:2] = 0.0
    m2[:, -2:] = 0.0
    return m1.reshape(1, _NPIX), m2.reshape(1, _NPIX)


def _rarity_body(x_ref, m1_ref, m2_ref, out_ref):
    m1 = m1_ref[...]
    m2 = m2_ref[...]
    tmins, tmaxs = [], []
    mv = None
    for b in range(_NB):
        x = x_ref[0, b] * m1
        tmin = jnp.min(x, axis=1, keepdims=True)
        tmax = jnp.max(x, axis=1, keepdims=True)
        tmins.append(tmin)
        tmaxs.append(tmax)
        rmax = (tmax - tmin) / (tmax - tmin + 1e-8) * 256.0
        mb = jnp.max(rmax, axis=(0, 1), keepdims=True)
        mv = mb if mv is None else jnp.maximum(mv, mb)
    inv_step = _BINS / mv

    BIG = jnp.float32(3.0e38)
    inv_n = jnp.float32(1.0 / _NPIX)
    for b in range(_NB):
        x = x_ref[0, b] * m1
        tmin, tmax = tmins[b], tmaxs[b]
        d = tmax - tmin + 1e-8
        y = (x - tmin) / d * 256.0
        b1 = jnp.minimum(jnp.floor(y * inv_step), float(_BINS - 1))
        b2 = jnp.clip((y / 256.0 * float(_BINS - 1)).astype(jnp.int32),
                      0, _BINS - 1)
        h1, c2, c2i = [], [], []
        for j in range(_BINS):
            f1 = jnp.where(b1 == float(j), 1.0, 0.0)
            f2 = jnp.where(b2 == j, 1.0, 0.0)
            h1.append(jnp.sum(f1, axis=1, keepdims=True))
            c2.append(jnp.sum(f2, axis=1, keepdims=True))
            c2i.append(jnp.sum(f2 * m2, axis=1, keepdims=True))
        hv = [-jnp.log(h / float(_NPIX) + 1e-4) for h in h1]
        occ = [c > 0.0 for c in c2]
        dmin = functools.reduce(jnp.minimum,
                                [jnp.where(o, v, BIG) for o, v in zip(occ, hv)])
        dmax = functools.reduce(jnp.maximum,
                                [jnp.where(o, v, -BIG) for o, v in zip(occ, hv)])
        dd = dmax - dmin + 1e-8
        g1 = [(v - dmin) / dd for v in hv]
        map_max = (dmax - dmin) / dd
        map_mean = functools.reduce(
            jnp.add, [c * g for c, g in zip(c2, g1)]) * inv_n
        w1 = (map_max - map_mean) ** 2
        g2 = [(g * w1) ** 2 for g in g1]
        ma = functools.reduce(jnp.maximum,
                              [jnp.where(o, v, -BIG) for o, v in zip(occ, g2)])
        me = functools.reduce(
            jnp.add, [c * g for c, g in zip(c2, g2)]) * inv_n
        w2 = (ma - me) ** 2
        g3 = [w2 * g for g in g2]
        t3min = functools.reduce(jnp.minimum,
                                 [jnp.where(o, v, BIG) for o, v in zip(occ, g3)])
        t3max = functools.reduce(jnp.maximum,
                                 [jnp.where(o, v, -BIG) for o, v in zip(occ, g3)])
        d3 = t3max - t3min + 1e-8
        g4 = [(v - t3min) / d3 for v in g3]
        g5 = [v * v for v in g4]
        occi = [c > 0.0 for c in c2i]
        ma2 = functools.reduce(jnp.maximum,
                               [jnp.where(o, v, 0.0) for o, v in zip(occi, g5)])
        me2 = functools.reduce(
            jnp.add, [c * g for c, g in zip(c2i, g5)]) * inv_n
        w3 = (ma2 - me2) ** 2
        g6 = [w3 * v for v in g5]
        acc = None
        for j in range(_BINS):
            contrib = jnp.where(b2 == j, g6[j], 0.0)
            acc = contrib if acc is None else acc + contrib
        s = jnp.sum(acc, axis=0, keepdims=True) * m2
        out_ref[0, b] = s


def _fuse_body(maps_ref, r_ref, rt_ref, groups_ref, sal_ref):
    rmat = r_ref[...]
    rtmat = rt_ref[...]
    inv_n = jnp.float32(1.0 / (_R * _R))
    sal = None
    for g in range(_NGROUPS):
        gsum = None
        for li in range(2):
            layer = 2 * g + li
            for b in range(_NB):
                m = maps_ref[layer, b]
                t = jnp.dot(rmat, m, preferred_element_type=jnp.float32)
                u = jnp.dot(t, rtmat, preferred_element_type=jnp.float32)
                mn = jnp.min(u, axis=(0, 1), keepdims=True)
                mx = jnp.max(u, axis=(0, 1), keepdims=True)
                t1 = (u - mn) / (mx - mn + 1e-8)
                mn2 = jnp.min(t1, axis=(0, 1), keepdims=True)
                mx2 = jnp.max(t1, axis=(0, 1), keepdims=True)
                t2 = (t1 - mn2) / (mx2 - mn2 + 1e-8)
                mx3 = jnp.max(t2, axis=(0, 1), keepdims=True)
                mean = jnp.sum(t2, axis=(0, 1), keepdims=True) * inv_n
                w1 = (mx3 - mean) ** 2
                t3 = w1 * t2
                gsum = t3 if gsum is None else gsum + t3
        gmn = jnp.min(gsum, axis=(0, 1), keepdims=True)
        gmx = jnp.max(gsum, axis=(0, 1), keepdims=True)
        gmap = (gsum - gmn) / (gmx - gmn + 1e-8) * 255.0
        groups_ref[g] = gmap
        sal = gmap if sal is None else sal + gmap
    smn = jnp.min(sal, axis=(0, 1), keepdims=True)
    smx = jnp.max(sal, axis=(0, 1), keepdims=True)
    s1 = (sal - smn) / (smx - smn + 1e-8)
    e = jnp.exp(s1)
    emn = jnp.min(e, axis=(0, 1), keepdims=True)
    emx = jnp.max(e, axis=(0, 1), keepdims=True)
    sal_ref[...] = (e - emn) / (emx - emn + 1e-8)


def kernel(layer_output):
    x = layer_output.reshape(17, _NB, _NC, _NPIX)
    m1, m2 = _border_masks()
    m1 = jnp.asarray(m1)
    m2 = jnp.asarray(m2)

    maps = pl.pallas_call(
        _rarity_body,
        grid=(_NLAYERS,),
        in_specs=[
            pl.BlockSpec((1, _NB, _NC, _NPIX),
                         lambda i: (3 * (i // 2) + 3 + i % 2, 0, 0, 0)),
            pl.BlockSpec((1, _NPIX), lambda i: (0, 0)),
            pl.BlockSpec((1, _NPIX), lambda i: (0, 0)),
        ],
        out_specs=pl.BlockSpec((1, _NB, _NPIX), lambda i: (i, 0, 0)),
        out_shape=jax.ShapeDtypeStruct((_NLAYERS, _NB, _NPIX), jnp.float32),
    )(x, m1, m2)

    maps = maps.reshape(_NLAYERS, _NB, _A, _A)
    eye = jnp.eye(_A, dtype=jnp.float32)
    rmat = jax.image.resize(eye, (_R, _A), method="bilinear")

    groups, sal = pl.pallas_call(
        _fuse_body,
        out_shape=(
            jax.ShapeDtypeStruct((_NGROUPS, _R, _R), jnp.float32),
            jax.ShapeDtypeStruct((_R, _R), jnp.float32),
        ),
    )(maps, rmat, rmat.T)

    return sal.reshape(1, _R, _R), groups.reshape(1, _NGROUPS, _R, _R)


# speedup vs baseline: 1458.5255x; 1458.5255x over previous
import functools

import jax
import jax.numpy as jnp
import numpy as np
from jax.experimental import pallas as pl

_NB, _NC, _A = 4, 192, 56
_NPIX = _A * _A
_BINS = 11
_NLAYERS = 10
_NGROUPS = 5
_R = 240


def _border_masks():
    m1 = np.ones((_A, _A), np.float32)
    m1[0, :] = 0.0
    m1[-1, :] = 0.0
    m1[:, 0] = 0.0
    m1[:, -1] = 0.0
    m2 = np.ones((_A, _A), np.float32)
    m2[0:2, :] = 0.0
    m2[-2:, :] = 0.0
    m2[:, 0:2] = 0.0
    m2[:, -2:] = 0.0
    return m1.reshape(1, _NPIX), m2.reshape(1, _NPIX)


def _rarity_body(x_ref, m1_ref, m2_ref, out_ref):
    m1 = m1_ref[...]
    m2 = m2_ref[...]
    tmins, tmaxs = [], []
    mv = None
    for b in range(_NB):
        x = x_ref[0, b] * m1
        tmin = jnp.min(x, axis=1, keepdims=True)
        tmax = jnp.max(x, axis=1, keepdims=True)
        tmins.append(tmin)
        tmaxs.append(tmax)
        rmax = (tmax - tmin) / (tmax - tmin + 1e-8) * 256.0
        mb = jnp.max(rmax, axis=(0, 1), keepdims=True)
        mv = mb if mv is None else jnp.maximum(mv, mb)
    inv_step = _BINS / mv

    BIG = jnp.float32(3.0e38)
    inv_n = jnp.float32(1.0 / _NPIX)
    for b in range(_NB):
        x = x_ref[0, b] * m1
        tmin, tmax = tmins[b], tmaxs[b]
        d = tmax - tmin + 1e-8
        y = (x - tmin) / d * 256.0
        b1 = jnp.minimum(jnp.floor(y * inv_step), float(_BINS - 1))
        b2 = jnp.clip((y / 256.0 * float(_BINS - 1)).astype(jnp.int32),
                      0, _BINS - 1)
        h1, c2, c2i = [], [], []
        for j in range(_BINS):
            f1 = jnp.where(b1 == float(j), 1.0, 0.0)
            f2 = jnp.where(b2 == j, 1.0, 0.0)
            h1.append(jnp.sum(f1, axis=1, keepdims=True))
            c2.append(jnp.sum(f2, axis=1, keepdims=True))
            c2i.append(jnp.sum(f2 * m2, axis=1, keepdims=True))
        hv = [-jnp.log(h / float(_NPIX) + 1e-4) for h in h1]
        occ = [c > 0.0 for c in c2]
        dmin = functools.reduce(jnp.minimum,
                                [jnp.where(o, v, BIG) for o, v in zip(occ, hv)])
        dmax = functools.reduce(jnp.maximum,
                                [jnp.where(o, v, -BIG) for o, v in zip(occ, hv)])
        dd = dmax - dmin + 1e-8
        g1 = [(v - dmin) / dd for v in hv]
        map_max = (dmax - dmin) / dd
        map_mean = functools.reduce(
            jnp.add, [c * g for c, g in zip(c2, g1)]) * inv_n
        w1 = (map_max - map_mean) ** 2
        g2 = [(g * w1) ** 2 for g in g1]
        ma = functools.reduce(jnp.maximum,
                              [jnp.where(o, v, -BIG) for o, v in zip(occ, g2)])
        me = functools.reduce(
            jnp.add, [c * g for c, g in zip(c2, g2)]) * inv_n
        w2 = (ma - me) ** 2
        g3 = [w2 * g for g in g2]
        t3min = functools.reduce(jnp.minimum,
                                 [jnp.where(o, v, BIG) for o, v in zip(occ, g3)])
        t3max = functools.reduce(jnp.maximum,
                                 [jnp.where(o, v, -BIG) for o, v in zip(occ, g3)])
        d3 = t3max - t3min + 1e-8
        g4 = [(v - t3min) / d3 for v in g3]
        g5 = [v * v for v in g4]
        occi = [c > 0.0 for c in c2i]
        ma2 = functools.reduce(jnp.maximum,
                               [jnp.where(o, v, 0.0) for o, v in zip(occi, g5)])
        me2 = functools.reduce(
            jnp.add, [c * g for c, g in zip(c2i, g5)]) * inv_n
        w3 = (ma2 - me2) ** 2
        g6 = [w3 * v for v in g5]
        acc = None
        for j in range(_BINS):
            contrib = jnp.where(b2 == j, g6[j], 0.0)
            acc = contrib if acc is None else acc + contrib
        s = jnp.sum(acc, axis=0, keepdims=True) * m2
        out_ref[0, pl.ds(b, 1)] = s


def _fuse_body(maps_ref, r_ref, rt_ref, groups_ref, sal_ref):
    rmat = r_ref[...]
    rtmat = rt_ref[...]
    inv_n = jnp.float32(1.0 / (_R * _R))
    sal = None
    for g in range(_NGROUPS):
        gsum = None
        for li in range(2):
            layer = 2 * g + li
            for b in range(_NB):
                m = maps_ref[layer, b]
                t = jnp.dot(rmat, m, preferred_element_type=jnp.float32)
                u = jnp.dot(t, rtmat, preferred_element_type=jnp.float32)
                mn = jnp.min(u, axis=(0, 1), keepdims=True)
                mx = jnp.max(u, axis=(0, 1), keepdims=True)
                t1 = (u - mn) / (mx - mn + 1e-8)
                mn2 = jnp.min(t1, axis=(0, 1), keepdims=True)
                mx2 = jnp.max(t1, axis=(0, 1), keepdims=True)
                t2 = (t1 - mn2) / (mx2 - mn2 + 1e-8)
                mx3 = jnp.max(t2, axis=(0, 1), keepdims=True)
                mean = jnp.sum(t2, axis=(0, 1), keepdims=True) * inv_n
                w1 = (mx3 - mean) ** 2
                t3 = w1 * t2
                gsum = t3 if gsum is None else gsum + t3
        gmn = jnp.min(gsum, axis=(0, 1), keepdims=True)
        gmx = jnp.max(gsum, axis=(0, 1), keepdims=True)
        gmap = (gsum - gmn) / (gmx - gmn + 1e-8) * 255.0
        groups_ref[g] = gmap
        sal = gmap if sal is None else sal + gmap
    smn = jnp.min(sal, axis=(0, 1), keepdims=True)
    smx = jnp.max(sal, axis=(0, 1), keepdims=True)
    s1 = (sal - smn) / (smx - smn + 1e-8)
    e = jnp.exp(s1)
    emn = jnp.min(e, axis=(0, 1), keepdims=True)
    emx = jnp.max(e, axis=(0, 1), keepdims=True)
    sal_ref[...] = (e - emn) / (emx - emn + 1e-8)


def kernel(layer_output):
    x = layer_output.reshape(17, _NB, _NC, _NPIX)
    m1, m2 = _border_masks()
    m1 = jnp.asarray(m1)
    m2 = jnp.asarray(m2)

    maps = pl.pallas_call(
        _rarity_body,
        grid=(_NLAYERS,),
        in_specs=[
            pl.BlockSpec((1, _NB, _NC, _NPIX),
                         lambda i: (3 * (i // 2) + 3 + i % 2, 0, 0, 0)),
            pl.BlockSpec((1, _NPIX), lambda i: (0, 0)),
            pl.BlockSpec((1, _NPIX), lambda i: (0, 0)),
        ],
        out_specs=pl.BlockSpec((1, _NB, _NPIX), lambda i: (i, 0, 0)),
        out_shape=jax.ShapeDtypeStruct((_NLAYERS, _NB, _NPIX), jnp.float32),
    )(x, m1, m2)

    maps = maps.reshape(_NLAYERS, _NB, _A, _A)
    eye = jnp.eye(_A, dtype=jnp.float32)
    rmat = jax.image.resize(eye, (_R, _A), method="bilinear")

    groups, sal = pl.pallas_call(
        _fuse_body,
        out_shape=(
            jax.ShapeDtypeStruct((_NGROUPS, _R, _R), jnp.float32),
            jax.ShapeDtypeStruct((_R, _R), jnp.float32),
        ),
    )(maps, rmat, rmat.T)

    return sal.reshape(1, _R, _R), groups.reshape(1, _NGROUPS, _R, _R)
